# baseline (speedup 1.0000x reference)
import functools

import jax
import jax.numpy as jnp
from jax.experimental import pallas as pl
from jax.experimental.pallas import tpu as pltpu

_K = 8


def _dot(a, b):
    return jax.lax.dot_general(
        a, b, (((1,), (0,)), ((), ())),
        preferred_element_type=jnp.float32)


def _lanref_kernel(tgt_ref, box_hbm, phr_ref,
                   w1s_ref, b1s_ref, w2s_ref, b2s_ref,
                   w1ts_hbm, b1ts_ref, w2ts_ref, b2ts_ref,
                   w1tr_hbm, b1tr_ref, w2tr_ref, b2tr_ref,
                   sim_out, det_out, reg_out,
                   w1ts_ref, w1tr_ref, box_ref, sem,
                   *, B, P, N, D_REC, D_PHR):
    f32 = jnp.float32

    cp_ts = pltpu.make_async_copy(w1ts_hbm, w1ts_ref, sem.at[0])
    cp_tr = pltpu.make_async_copy(w1tr_hbm, w1tr_ref, sem.at[1])
    cp_bx = pltpu.make_async_copy(box_hbm, box_ref, sem.at[2])
    cp_bx.start()
    cp_ts.start()
    cp_tr.start()

    phr2d = phr_ref[...].reshape(B * P, D_PHR)
    rowid = [jnp.full((1, 1), tgt_ref[b] + b * P, jnp.int32) for b in range(B)]
    rowid = jnp.concatenate(rowid, axis=0)
    iota_bp = jax.lax.broadcasted_iota(jnp.int32, (B, B * P), 1)
    oh_p = (iota_bp == rowid).astype(f32)
    phr_t = _dot(oh_p, phr2d)

    cp_bx.wait()
    hb_all = _dot(box_ref[...].reshape(B * N, D_REC),
                  w1s_ref[0:D_REC, :])
    hp_sim = _dot(phr_t, w1s_ref[D_REC:, :])
    hp_sim_rep = jnp.concatenate(
        [jnp.broadcast_to(hp_sim[b:b + 1, :], (N, hp_sim.shape[1]))
         for b in range(B)], axis=0)
    h = hb_all + hp_sim_rep + b1s_ref[...]
    h = jnp.where(h > 0, h, 0.01 * h)
    sim_all = jnp.dot(h, w2s_ref[...],
                      preferred_element_type=f32) + b2s_ref[...]

    sim_nb = jnp.concatenate(
        [sim_all[b * N:(b + 1) * N, :] for b in range(B)], axis=1)
    sim_mat = jnp.transpose(sim_nb)
    sim_out[...] = sim_mat

    lowmask = (jax.lax.broadcasted_iota(jnp.int32, (N, N), 0) <
               jax.lax.broadcasted_iota(jnp.int32, (N, N), 1))
    iota_k1 = jax.lax.broadcasted_iota(jnp.int32, (_K, 1), 0)
    onehots, topvs = [], []
    for b in range(B):
        s_col = sim_nb[:, b:b + 1]
        s_row = sim_mat[b:b + 1, :]
        beats = (s_col > s_row) | ((s_col == s_row) & lowmask)
        rank = jnp.sum(beats.astype(jnp.int32),
                       axis=0, keepdims=True)
        onehots.append((jnp.broadcast_to(rank, (_K, N)) ==
                        iota_k1).astype(f32))
        topvs.append(_dot(onehots[b], s_col))

    cp_ts.wait()
    cp_tr.wait()
    w1ts_box, w1ts_phr = w1ts_ref[0:D_REC, :], w1ts_ref[D_REC:, :]
    w1tr_box, w1tr_phr = w1tr_ref[0:D_REC, :], w1tr_ref[D_REC:, :]
    hp_ts = _dot(phr_t, w1ts_phr)
    hp_tr = _dot(phr_t, w1tr_phr)

    box_top = jnp.concatenate(
        [_dot(onehots[b], box_ref[b]) for b in range(B)], axis=0)
    hp_ts_rep = jnp.concatenate(
        [jnp.broadcast_to(hp_ts[b:b + 1, :], (_K, hp_ts.shape[1]))
         for b in range(B)], axis=0)
    hp_tr_rep = jnp.concatenate(
        [jnp.broadcast_to(hp_tr[b:b + 1, :], (_K, hp_tr.shape[1]))
         for b in range(B)], axis=0)

    h2s = _dot(box_top, w1ts_box) + hp_ts_rep + b1ts_ref[...]
    h2s = jnp.where(h2s > 0, h2s, 0.01 * h2s)
    sim2 = _dot(h2s, w2ts_ref[...]) + b2ts_ref[...]

    h2r = _dot(box_top, w1tr_box) + hp_tr_rep + b1tr_ref[...]
    h2r = jnp.where(h2r > 0, h2r, 0.01 * h2r)
    reg2 = _dot(h2r, w2tr_ref[...]) + b2tr_ref[...]
    for b in range(B):
        reg_out[b] = reg2[b * _K:(b + 1) * _K, :]

    topv_all = jnp.concatenate(topvs, axis=0)
    fused = sim2 * topv_all
    det_rows = []
    for b in range(B):
        fused_row = jnp.transpose(fused[b * _K:(b + 1) * _K, :])
        scattered = _dot(fused_row, onehots[b])
        selmask = _dot(jnp.ones((1, _K), f32), onehots[b])
        det_rows.append(jnp.where(selmask > 0, scattered, -1e9))
    det_out[...] = jnp.concatenate(det_rows, axis=0)


def kernel(box_features, phrase_embed, target_id,
           W1_sim, b1_sim, W2_sim, b2_sim,
           W1_reg, b1_reg, W2_reg, b2_reg,
           W1_sim_topN, b1_sim_topN, W2_sim_topN, b2_sim_topN,
           W1_reg_topN, b1_reg_topN, W2_reg_topN, b2_reg_topN):
    del W1_reg, b1_reg, W2_reg, b2_reg
    B, N, D_REC = box_features.shape
    _, P, D_PHR = phrase_embed.shape
    f32 = jnp.float32

    vm = pl.BlockSpec(memory_space=pltpu.VMEM)
    anymem = pl.BlockSpec(memory_space=pltpu.MemorySpace.HBM)
    HID = W1_sim.shape[1]
    SIM_IN = W1_sim.shape[0]
    sim_t, det, reg = pl.pallas_call(
        functools.partial(_lanref_kernel, B=B, P=P, N=N, D_REC=D_REC,
                          D_PHR=D_PHR),
        in_specs=([pl.BlockSpec(memory_space=pltpu.SMEM)] + [anymem] +
                  [vm] * 5 + [anymem] + [vm] * 3 + [anymem] + [vm] * 3),
        out_specs=[vm, vm, vm],
        scratch_shapes=[pltpu.VMEM((SIM_IN, HID), f32),
                        pltpu.VMEM((SIM_IN, HID), f32),
                        pltpu.VMEM((B, N, D_REC), f32),
                        pltpu.SemaphoreType.DMA((3,))],
        out_shape=[
            jax.ShapeDtypeStruct((B, N), f32),
            jax.ShapeDtypeStruct((B, N), f32),
            jax.ShapeDtypeStruct((B, _K, 6), f32),
        ],
    )(target_id, box_features, phrase_embed,
      W1_sim, b1_sim.reshape(1, -1), W2_sim, b2_sim.reshape(1, -1),
      W1_sim_topN, b1_sim_topN.reshape(1, -1), W2_sim_topN,
      b2_sim_topN.reshape(1, -1),
      W1_reg_topN, b1_reg_topN.reshape(1, -1), W2_reg_topN,
      b2_reg_topN.reshape(1, -1))
    return sim_t, det, reg


# speedup vs baseline: 1.0680x; 1.0680x over previous
import functools

import jax
import jax.numpy as jnp
from jax.experimental import pallas as pl
from jax.experimental.pallas import tpu as pltpu

_K = 8


def _dot(a, b):
    return jax.lax.dot_general(
        a, b, (((1,), (0,)), ((), ())),
        preferred_element_type=jnp.float32)


def _lanref_kernel(tgt_ref, box_ref, phr_ref,
                   w1s_ref, b1s_ref, w2s_ref, b2s_ref,
                   w1ts_hbm, b1ts_ref, w2ts_ref, b2ts_ref,
                   w1tr_hbm, b1tr_ref, w2tr_ref, b2tr_ref,
                   sim_out, det_out, reg_out,
                   w1ts_ref, w1tr_ref, sem,
                   *, B, P, N, D_REC, D_PHR):
    f32 = jnp.float32

    cp_ts = pltpu.make_async_copy(w1ts_hbm, w1ts_ref, sem.at[0])
    cp_tr = pltpu.make_async_copy(w1tr_hbm, w1tr_ref, sem.at[1])
    cp_ts.start()
    cp_tr.start()

    phr2d = phr_ref[...].reshape(B * P, D_PHR)
    rowid = [jnp.full((1, 1), tgt_ref[b] + b * P, jnp.int32) for b in range(B)]
    rowid = jnp.concatenate(rowid, axis=0)
    iota_bp = jax.lax.broadcasted_iota(jnp.int32, (B, B * P), 1)
    oh_p = (iota_bp == rowid).astype(f32)
    phr_t = _dot(oh_p, phr2d)

    hb_all = _dot(box_ref[...].reshape(B * N, D_REC),
                  w1s_ref[0:D_REC, :])
    hp_sim = _dot(phr_t, w1s_ref[D_REC:, :])
    hp_sim_rep = jnp.concatenate(
        [jnp.broadcast_to(hp_sim[b:b + 1, :], (N, hp_sim.shape[1]))
         for b in range(B)], axis=0)
    h = hb_all + hp_sim_rep + b1s_ref[...]
    h = jnp.where(h > 0, h, 0.01 * h)
    sim_all = jnp.dot(h, w2s_ref[...],
                      preferred_element_type=f32) + b2s_ref[...]

    sim_nb = jnp.concatenate(
        [sim_all[b * N:(b + 1) * N, :] for b in range(B)], axis=1)
    sim_mat = jnp.transpose(sim_nb)
    sim_out[...] = sim_mat

    lowmask = (jax.lax.broadcasted_iota(jnp.int32, (N, N), 0) <
               jax.lax.broadcasted_iota(jnp.int32, (N, N), 1))
    iota_k1 = jax.lax.broadcasted_iota(jnp.int32, (_K, 1), 0)
    onehots, topvs = [], []
    for b in range(B):
        s_col = sim_nb[:, b:b + 1]
        s_row = sim_mat[b:b + 1, :]
        beats = (s_col > s_row) | ((s_col == s_row) & lowmask)
        rank = jnp.sum(beats.astype(jnp.int32),
                       axis=0, keepdims=True)
        onehots.append((jnp.broadcast_to(rank, (_K, N)) ==
                        iota_k1).astype(f32))
        topvs.append(_dot(onehots[b], s_col))

    cp_ts.wait()
    cp_tr.wait()
    w1ts_box, w1ts_phr = w1ts_ref[0:D_REC, :], w1ts_ref[D_REC:, :]
    w1tr_box, w1tr_phr = w1tr_ref[0:D_REC, :], w1tr_ref[D_REC:, :]
    hp_ts = _dot(phr_t, w1ts_phr)
    hp_tr = _dot(phr_t, w1tr_phr)

    box_top = jnp.concatenate(
        [_dot(onehots[b], box_ref[b]) for b in range(B)], axis=0)
    hp_ts_rep = jnp.concatenate(
        [jnp.broadcast_to(hp_ts[b:b + 1, :], (_K, hp_ts.shape[1]))
         for b in range(B)], axis=0)
    hp_tr_rep = jnp.concatenate(
        [jnp.broadcast_to(hp_tr[b:b + 1, :], (_K, hp_tr.shape[1]))
         for b in range(B)], axis=0)

    h2s = _dot(box_top, w1ts_box) + hp_ts_rep + b1ts_ref[...]
    h2s = jnp.where(h2s > 0, h2s, 0.01 * h2s)
    sim2 = _dot(h2s, w2ts_ref[...]) + b2ts_ref[...]

    h2r = _dot(box_top, w1tr_box) + hp_tr_rep + b1tr_ref[...]
    h2r = jnp.where(h2r > 0, h2r, 0.01 * h2r)
    reg2 = _dot(h2r, w2tr_ref[...]) + b2tr_ref[...]
    for b in range(B):
        reg_out[b] = reg2[b * _K:(b + 1) * _K, :]

    topv_all = jnp.concatenate(topvs, axis=0)
    fused = sim2 * topv_all
    det_rows = []
    for b in range(B):
        fused_row = jnp.transpose(fused[b * _K:(b + 1) * _K, :])
        scattered = _dot(fused_row, onehots[b])
        selmask = _dot(jnp.ones((1, _K), f32), onehots[b])
        det_rows.append(jnp.where(selmask > 0, scattered, -1e9))
    det_out[...] = jnp.concatenate(det_rows, axis=0)


def kernel(box_features, phrase_embed, target_id,
           W1_sim, b1_sim, W2_sim, b2_sim,
           W1_reg, b1_reg, W2_reg, b2_reg,
           W1_sim_topN, b1_sim_topN, W2_sim_topN, b2_sim_topN,
           W1_reg_topN, b1_reg_topN, W2_reg_topN, b2_reg_topN):
    del W1_reg, b1_reg, W2_reg, b2_reg
    B, N, D_REC = box_features.shape
    _, P, D_PHR = phrase_embed.shape
    f32 = jnp.float32

    vm = pl.BlockSpec(memory_space=pltpu.VMEM)
    anymem = pl.BlockSpec(memory_space=pltpu.MemorySpace.HBM)
    HID = W1_sim.shape[1]
    SIM_IN = W1_sim.shape[0]
    sim_t, det, reg = pl.pallas_call(
        functools.partial(_lanref_kernel, B=B, P=P, N=N, D_REC=D_REC,
                          D_PHR=D_PHR),
        in_specs=([pl.BlockSpec(memory_space=pltpu.SMEM)] + [vm] * 6 +
                  [anymem] + [vm] * 3 + [anymem] + [vm] * 3),
        out_specs=[vm, vm, vm],
        scratch_shapes=[pltpu.VMEM((SIM_IN, HID), f32),
                        pltpu.VMEM((SIM_IN, HID), f32),
                        pltpu.SemaphoreType.DMA((2,))],
        out_shape=[
            jax.ShapeDtypeStruct((B, N), f32),
            jax.ShapeDtypeStruct((B, N), f32),
            jax.ShapeDtypeStruct((B, _K, 6), f32),
        ],
    )(target_id, box_features, phrase_embed,
      W1_sim, b1_sim.reshape(1, -1), W2_sim, b2_sim.reshape(1, -1),
      W1_sim_topN, b1_sim_topN.reshape(1, -1), W2_sim_topN,
      b2_sim_topN.reshape(1, -1),
      W1_reg_topN, b1_reg_topN.reshape(1, -1), W2_reg_topN,
      b2_reg_topN.reshape(1, -1))
    return sim_t, det, reg
